# baseline (speedup 1.0000x reference)
import functools

import jax
import jax.numpy as jnp
import numpy as np
from jax import lax
from jax.experimental import pallas as pl
from jax.experimental.pallas import tpu as pltpu
from jax.experimental.pallas import tpu_sc as plsc

N = 10000
E = 320000
D = 128
DEPTH = 5

NC = 2
NS = 16
NW = NC * NS
E_PER = E // NW
DEG_CH = 2000
EDGE_CH = 80
N_CHUNKS = E_PER // EDGE_CH
SP_CH = 128
NSETS = 3
SP_ITERS = E_PER // (NSETS * SP_CH)
SP_FULL = SP_ITERS * NSETS
SP_REM = E_PER - SP_FULL * SP_CH
ROWS_PER_TILE = N // NS

@functools.cache
def _make_deg():
    mesh = plsc.VectorSubcoreMesh(core_axis_name="c", subcore_axis_name="s",
                                  num_cores=NC, num_subcores=NS)
    return pl.kernel(
        _deg_body,
        out_type=jax.ShapeDtypeStruct((NC, N), jnp.float32),
        mesh=mesh,
        scratch_types=[
            pltpu.VMEM((E_PER,), jnp.int32),
            pltpu.VMEM((EDGE_CH,), jnp.float32),
            pltpu.VMEM_SHARED((N,), jnp.float32),
            pltpu.SemaphoreType.DMA,
        ],
        compiler_params=pltpu.CompilerParams(use_tc_tiling_on_sc=False),
    )


def _deg_body(ei_hbm, zeros1_hbm, out_hbm, didx_v, ones_v, deg_s, sem):
    cid = lax.axis_index("c")
    sid = lax.axis_index("s")
    wid = sid * NC + cid
    base = wid * E_PER

    def fill(i, _):
        ones_v[pl.ds(i * 16, 16)] = jnp.full((16,), 1.0, jnp.float32)
        return 0

    lax.fori_loop(0, EDGE_CH // 16, fill, 0)

    @pl.when(sid < 10)
    def _():
        pltpu.sync_copy(zeros1_hbm.at[pl.ds(sid * 1000, 1000)],
                        deg_s.at[pl.ds(sid * 1000, 1000)])

    pltpu.sync_copy(ei_hbm.at[1, pl.ds(base, E_PER)], didx_v)
    plsc.subcore_barrier()

    def chunk(c, _):
        pltpu.async_copy(
            ones_v, deg_s.at[didx_v.at[pl.ds(c * EDGE_CH, EDGE_CH)]], sem,
            add=True)
        return 0

    lax.fori_loop(0, N_CHUNKS, chunk, 0)
    pltpu.make_async_copy(ei_hbm.at[1, pl.ds(base, E_PER)], didx_v, sem).wait()
    plsc.subcore_barrier()

    @pl.when(sid < 10)
    def _():
        pltpu.sync_copy(deg_s.at[pl.ds(sid * 1000, 1000)],
                        out_hbm.at[cid, pl.ds(sid * 1000, 1000)])


@functools.cache
def _make_spmm():
    mesh = plsc.VectorSubcoreMesh(core_axis_name="c", subcore_axis_name="s",
                                  num_cores=NC, num_subcores=NS)
    return pl.kernel(
        _spmm_body,
        out_type=jax.ShapeDtypeStruct((NC, N, D), jnp.float32),
        mesh=mesh,
        scratch_types=(
            [pltpu.VMEM((SP_CH,), jnp.int32) for _ in range(NSETS)]
            + [pltpu.VMEM((SP_CH,), jnp.int32) for _ in range(NSETS)]
            + [pltpu.VMEM((SP_CH, D), jnp.float32) for _ in range(NSETS)]
            + [pltpu.VMEM_SHARED((N, D), jnp.float32)]
            + [pltpu.SemaphoreType.DMA for _ in range(2 * NSETS)]
        ),
        compiler_params=pltpu.CompilerParams(use_tc_tiling_on_sc=False),
    )


def _spmm_body(ei_hbm, g_hbm, zeros_hbm, out_hbm, *refs):
    sidx = refs[0:NSETS]
    didx = refs[NSETS:2 * NSETS]
    rows = refs[2 * NSETS:3 * NSETS]
    acc_s = refs[3 * NSETS]
    semg = refs[3 * NSETS + 1:3 * NSETS + 1 + NSETS]
    sems = refs[3 * NSETS + 1 + NSETS:3 * NSETS + 1 + 2 * NSETS]

    cid = lax.axis_index("c")
    sid = lax.axis_index("s")
    wid = sid * NC + cid
    base = wid * E_PER
    row0 = sid * ROWS_PER_TILE

    pltpu.sync_copy(zeros_hbm.at[pl.ds(row0, ROWS_PER_TILE)],
                    acc_s.at[pl.ds(row0, ROWS_PER_TILE)])
    plsc.subcore_barrier()

    def fire(s, e0):
        pltpu.sync_copy(ei_hbm.at[0, pl.ds(base + e0, SP_CH)], sidx[s])
        pltpu.sync_copy(ei_hbm.at[1, pl.ds(base + e0, SP_CH)], didx[s])
        pltpu.async_copy(g_hbm.at[sidx[s]], rows[s], semg[s])

    def wait_gather(s):
        pltpu.make_async_copy(zeros_hbm.at[pl.ds(0, SP_CH)], rows[s],
                              semg[s]).wait()

    def fire_scatter(s):
        pltpu.async_copy(rows[s], acc_s.at[didx[s]], sems[s], add=True)

    def drain_scatter(s):
        pltpu.make_async_copy(zeros_hbm.at[pl.ds(0, SP_CH)], rows[s],
                              sems[s]).wait()

    def ring(i, _):
        c0 = i * NSETS
        for s in range(NSETS):
            c = c0 + s

            @pl.when(i >= 1)
            def _():
                drain_scatter(s)

            fire(s, c * SP_CH)
            sw = (s - 2) % NSETS

            @pl.when(c0 + s >= 2)
            def _():
                wait_gather(sw)
                fire_scatter(sw)

        return 0

    lax.fori_loop(0, SP_ITERS, ring, 0)

    for c in (SP_FULL - 2, SP_FULL - 1):
        s = c % NSETS
        wait_gather(s)
        fire_scatter(s)
    if SP_REM:
        eL = base + SP_FULL * SP_CH
        sL = 0
        drain_scatter(sL)
        pltpu.sync_copy(ei_hbm.at[0, pl.ds(eL, SP_REM)],
                        sidx[sL].at[pl.ds(0, SP_REM)])
        pltpu.sync_copy(ei_hbm.at[1, pl.ds(eL, SP_REM)],
                        didx[sL].at[pl.ds(0, SP_REM)])
        pltpu.async_copy(g_hbm.at[sidx[sL].at[pl.ds(0, SP_REM)]],
                         rows[sL].at[pl.ds(0, SP_REM)], semg[sL]).wait()
        pltpu.sync_copy(rows[sL].at[pl.ds(0, SP_REM)],
                        acc_s.at[didx[sL].at[pl.ds(0, SP_REM)]], add=True)
        for s in range(1, NSETS):
            drain_scatter(s)
    else:
        for s in range(NSETS):
            drain_scatter(s)
    plsc.subcore_barrier()
    pltpu.sync_copy(acc_s.at[pl.ds(row0, ROWS_PER_TILE)],
                    out_hbm.at[cid, pl.ds(row0, ROWS_PER_TILE)])


def _scale_body(x_ref, w_ref, degp_ref, g_ref, dinv_ref):
    h0 = jnp.dot(x_ref[...], w_ref[...], preferred_element_type=jnp.float32)
    deg = jnp.sum(degp_ref[...], axis=0) + 1.0
    dinv = lax.rsqrt(deg)
    g_ref[...] = h0 * dinv[:, None]
    dinv_ref[...] = dinv


def _mlp_body(accp_ref, g_ref, dinv_ref, bgcn_ref, wenc_ref, benc_ref,
              wmu_ref, bmu_ref, wstd_ref, bstd_ref, eps_ref,
              xn_ref, mu_ref, std_ref):
    acc = accp_ref[0] + accp_ref[1]
    h = dinv_ref[...][:, None] * (acc + g_ref[...]) + bgcn_ref[...][None, :]
    for i in range(DEPTH):
        z = jnp.dot(h, wenc_ref[i], preferred_element_type=jnp.float32)
        h = jax.nn.sigmoid(z + benc_ref[i][None, :])
    mu = jnp.dot(h, wmu_ref[...], preferred_element_type=jnp.float32)
    mu = mu + bmu_ref[...][None, :]
    s = jnp.dot(h, wstd_ref[...], preferred_element_type=jnp.float32)
    s = s + bstd_ref[...][None, :] - 5.0
    std = jnp.maximum(s, 0.0) + jnp.log1p(jnp.exp(-jnp.abs(s)))
    mu_ref[...] = mu
    std_ref[...] = std
    xn_ref[...] = mu + std * eps_ref[...]


_R = 1024
_G = (N + _R - 1) // _R


@functools.cache
def _eps_np():
    with jax.ensure_compile_time_eval():
        with jax.default_device(jax.devices("cpu")[0]):
            return np.asarray(jax.random.uniform(
                jax.random.key(42), (N, D), dtype=jnp.float32))


_ZEROS_ND = np.zeros((N, D), np.float32)
_ZEROS_N = np.zeros((N,), np.float32)


def kernel(x, edge_index, W_gcn, b_gcn, W_enc, b_enc, W_mu, b_mu, W_std,
           b_std):
    deg_parts = _make_deg()(edge_index, _ZEROS_N)

    g, dinv = pl.pallas_call(
        _scale_body,
        grid=(_G,),
        in_specs=[
            pl.BlockSpec((_R, D), lambda i: (i, 0)),
            pl.BlockSpec((D, D), lambda i: (0, 0)),
            pl.BlockSpec((NC, _R), lambda i: (0, i)),
        ],
        out_specs=[
            pl.BlockSpec((_R, D), lambda i: (i, 0)),
            pl.BlockSpec((_R,), lambda i: (i,)),
        ],
        out_shape=[
            jax.ShapeDtypeStruct((N, D), jnp.float32),
            jax.ShapeDtypeStruct((N,), jnp.float32),
        ],
    )(x, W_gcn, deg_parts)

    acc_parts = _make_spmm()(edge_index, g, _ZEROS_ND)

    eps = _eps_np()

    x_new, mu, std = pl.pallas_call(
        _mlp_body,
        grid=(_G,),
        in_specs=[
            pl.BlockSpec((NC, _R, D), lambda i: (0, i, 0)),
            pl.BlockSpec((_R, D), lambda i: (i, 0)),
            pl.BlockSpec((_R,), lambda i: (i,)),
            pl.BlockSpec((D,), lambda i: (0,)),
            pl.BlockSpec((DEPTH, D, D), lambda i: (0, 0, 0)),
            pl.BlockSpec((DEPTH, D), lambda i: (0, 0)),
            pl.BlockSpec((D, D), lambda i: (0, 0)),
            pl.BlockSpec((D,), lambda i: (0,)),
            pl.BlockSpec((D, D), lambda i: (0, 0)),
            pl.BlockSpec((D,), lambda i: (0,)),
            pl.BlockSpec((_R, D), lambda i: (i, 0)),
        ],
        out_specs=[
            pl.BlockSpec((_R, D), lambda i: (i, 0)),
            pl.BlockSpec((_R, D), lambda i: (i, 0)),
            pl.BlockSpec((_R, D), lambda i: (i, 0)),
        ],
        out_shape=[
            jax.ShapeDtypeStruct((N, D), jnp.float32),
            jax.ShapeDtypeStruct((N, D), jnp.float32),
            jax.ShapeDtypeStruct((N, D), jnp.float32),
        ],
    )(acc_parts, g, dinv, b_gcn, W_enc, b_enc, W_mu, b_mu, W_std, b_std, eps)

    return (x_new, mu, std)


# speedup vs baseline: 1.1336x; 1.1336x over previous
import functools

import jax
import jax.numpy as jnp
import numpy as np
from jax import lax
from jax.experimental import pallas as pl
from jax.experimental.pallas import tpu as pltpu
from jax.experimental.pallas import tpu_sc as plsc

N = 10000
E = 320000
D = 128
DEPTH = 5

NC = 2
NS = 16
NW = NC * NS
E_PER = E // NW
DEG_CH = 2000
EDGE_CH = 80
N_CHUNKS = E_PER // EDGE_CH
SP_CH = 96
NSETS = 4
SP_ITERS = E_PER // (NSETS * SP_CH)
SP_FULL = SP_ITERS * NSETS
SP_REM = E_PER - SP_FULL * SP_CH
ROWS_PER_TILE = N // NS

@functools.cache
def _make_deg():
    mesh = plsc.VectorSubcoreMesh(core_axis_name="c", subcore_axis_name="s",
                                  num_cores=NC, num_subcores=NS)
    return pl.kernel(
        _deg_body,
        out_type=jax.ShapeDtypeStruct((NC, N), jnp.float32),
        mesh=mesh,
        scratch_types=[
            pltpu.VMEM((E_PER,), jnp.int32),
            pltpu.VMEM((EDGE_CH,), jnp.float32),
            pltpu.VMEM_SHARED((N,), jnp.float32),
            pltpu.SemaphoreType.DMA,
        ],
        compiler_params=pltpu.CompilerParams(use_tc_tiling_on_sc=False),
    )


def _deg_body(ei_hbm, zeros1_hbm, out_hbm, didx_v, ones_v, deg_s, sem):
    cid = lax.axis_index("c")
    sid = lax.axis_index("s")
    wid = sid * NC + cid
    base = wid * E_PER

    def fill(i, _):
        ones_v[pl.ds(i * 16, 16)] = jnp.full((16,), 1.0, jnp.float32)
        return 0

    lax.fori_loop(0, EDGE_CH // 16, fill, 0)

    @pl.when(sid < 10)
    def _():
        pltpu.sync_copy(zeros1_hbm.at[pl.ds(sid * 1000, 1000)],
                        deg_s.at[pl.ds(sid * 1000, 1000)])

    pltpu.sync_copy(ei_hbm.at[1, pl.ds(base, E_PER)], didx_v)
    plsc.subcore_barrier()

    def chunk(c, _):
        pltpu.async_copy(
            ones_v, deg_s.at[didx_v.at[pl.ds(c * EDGE_CH, EDGE_CH)]], sem,
            add=True)
        return 0

    lax.fori_loop(0, N_CHUNKS, chunk, 0)
    pltpu.make_async_copy(ei_hbm.at[1, pl.ds(base, E_PER)], didx_v, sem).wait()
    plsc.subcore_barrier()

    @pl.when(sid < 10)
    def _():
        pltpu.sync_copy(deg_s.at[pl.ds(sid * 1000, 1000)],
                        out_hbm.at[cid, pl.ds(sid * 1000, 1000)])


@functools.cache
def _make_spmm():
    mesh = plsc.VectorSubcoreMesh(core_axis_name="c", subcore_axis_name="s",
                                  num_cores=NC, num_subcores=NS)
    return pl.kernel(
        _spmm_body,
        out_type=jax.ShapeDtypeStruct((NC, N, D), jnp.float32),
        mesh=mesh,
        scratch_types=(
            [pltpu.VMEM((SP_CH,), jnp.int32) for _ in range(NSETS)]
            + [pltpu.VMEM((SP_CH,), jnp.int32) for _ in range(NSETS)]
            + [pltpu.VMEM((SP_CH, D), jnp.float32) for _ in range(NSETS)]
            + [pltpu.VMEM_SHARED((N, D), jnp.float32)]
            + [pltpu.SemaphoreType.DMA for _ in range(2 * NSETS)]
        ),
        compiler_params=pltpu.CompilerParams(use_tc_tiling_on_sc=False),
    )


def _spmm_body(ei_hbm, g_hbm, zeros_hbm, out_hbm, *refs):
    sidx = refs[0:NSETS]
    didx = refs[NSETS:2 * NSETS]
    rows = refs[2 * NSETS:3 * NSETS]
    acc_s = refs[3 * NSETS]
    semg = refs[3 * NSETS + 1:3 * NSETS + 1 + NSETS]
    sems = refs[3 * NSETS + 1 + NSETS:3 * NSETS + 1 + 2 * NSETS]

    cid = lax.axis_index("c")
    sid = lax.axis_index("s")
    wid = sid * NC + cid
    base = wid * E_PER
    row0 = sid * ROWS_PER_TILE

    pltpu.sync_copy(zeros_hbm.at[pl.ds(row0, ROWS_PER_TILE)],
                    acc_s.at[pl.ds(row0, ROWS_PER_TILE)])
    plsc.subcore_barrier()

    def fire(s, e0):
        pltpu.sync_copy(ei_hbm.at[0, pl.ds(base + e0, SP_CH)], sidx[s])
        pltpu.sync_copy(ei_hbm.at[1, pl.ds(base + e0, SP_CH)], didx[s])
        pltpu.async_copy(g_hbm.at[sidx[s]], rows[s], semg[s])

    def wait_gather(s):
        pltpu.make_async_copy(zeros_hbm.at[pl.ds(0, SP_CH)], rows[s],
                              semg[s]).wait()

    def fire_scatter(s):
        pltpu.async_copy(rows[s], acc_s.at[didx[s]], sems[s], add=True)

    def drain_scatter(s):
        pltpu.make_async_copy(zeros_hbm.at[pl.ds(0, SP_CH)], rows[s],
                              sems[s]).wait()

    def ring(i, _):
        c0 = i * NSETS
        for s in range(NSETS):
            c = c0 + s

            @pl.when(i >= 1)
            def _():
                drain_scatter(s)

            fire(s, c * SP_CH)
            sw = (s - 2) % NSETS

            @pl.when(c0 + s >= 2)
            def _():
                wait_gather(sw)
                fire_scatter(sw)

        return 0

    lax.fori_loop(0, SP_ITERS, ring, 0)

    for c in (SP_FULL - 2, SP_FULL - 1):
        s = c % NSETS
        wait_gather(s)
        fire_scatter(s)
    if SP_REM:
        eL = base + SP_FULL * SP_CH
        sL = 0
        drain_scatter(sL)
        pltpu.sync_copy(ei_hbm.at[0, pl.ds(eL, SP_REM)],
                        sidx[sL].at[pl.ds(0, SP_REM)])
        pltpu.sync_copy(ei_hbm.at[1, pl.ds(eL, SP_REM)],
                        didx[sL].at[pl.ds(0, SP_REM)])
        pltpu.async_copy(g_hbm.at[sidx[sL].at[pl.ds(0, SP_REM)]],
                         rows[sL].at[pl.ds(0, SP_REM)], semg[sL]).wait()
        pltpu.sync_copy(rows[sL].at[pl.ds(0, SP_REM)],
                        acc_s.at[didx[sL].at[pl.ds(0, SP_REM)]], add=True)
        for s in range(1, NSETS):
            drain_scatter(s)
    else:
        for s in range(NSETS):
            drain_scatter(s)
    plsc.subcore_barrier()
    pltpu.sync_copy(acc_s.at[pl.ds(row0, ROWS_PER_TILE)],
                    out_hbm.at[cid, pl.ds(row0, ROWS_PER_TILE)])


def _scale_body(x_ref, w_ref, degp_ref, g_ref, dinv_ref):
    h0 = jnp.dot(x_ref[...], w_ref[...], preferred_element_type=jnp.float32)
    deg = jnp.sum(degp_ref[...], axis=0) + 1.0
    dinv = lax.rsqrt(deg)
    g_ref[...] = h0 * dinv[:, None]
    dinv_ref[...] = dinv


def _mlp_body(accp_ref, g_ref, dinv_ref, bgcn_ref, wenc_ref, benc_ref,
              wmu_ref, bmu_ref, wstd_ref, bstd_ref, eps_ref,
              xn_ref, mu_ref, std_ref):
    acc = accp_ref[0] + accp_ref[1]
    h = dinv_ref[...][:, None] * (acc + g_ref[...]) + bgcn_ref[...][None, :]
    for i in range(DEPTH):
        z = jnp.dot(h, wenc_ref[i], preferred_element_type=jnp.float32)
        h = jax.nn.sigmoid(z + benc_ref[i][None, :])
    mu = jnp.dot(h, wmu_ref[...], preferred_element_type=jnp.float32)
    mu = mu + bmu_ref[...][None, :]
    s = jnp.dot(h, wstd_ref[...], preferred_element_type=jnp.float32)
    s = s + bstd_ref[...][None, :] - 5.0
    std = jnp.maximum(s, 0.0) + jnp.log1p(jnp.exp(-jnp.abs(s)))
    mu_ref[...] = mu
    std_ref[...] = std
    xn_ref[...] = mu + std * eps_ref[...]


_R = 1024
_G = (N + _R - 1) // _R


@functools.cache
def _eps_np():
    with jax.ensure_compile_time_eval():
        with jax.default_device(jax.devices("cpu")[0]):
            return np.asarray(jax.random.uniform(
                jax.random.key(42), (N, D), dtype=jnp.float32))


_ZEROS_ND = np.zeros((N, D), np.float32)
_ZEROS_N = np.zeros((N,), np.float32)


def kernel(x, edge_index, W_gcn, b_gcn, W_enc, b_enc, W_mu, b_mu, W_std,
           b_std):
    deg_parts = _make_deg()(edge_index, _ZEROS_N)

    g, dinv = pl.pallas_call(
        _scale_body,
        grid=(_G,),
        in_specs=[
            pl.BlockSpec((_R, D), lambda i: (i, 0)),
            pl.BlockSpec((D, D), lambda i: (0, 0)),
            pl.BlockSpec((NC, _R), lambda i: (0, i)),
        ],
        out_specs=[
            pl.BlockSpec((_R, D), lambda i: (i, 0)),
            pl.BlockSpec((_R,), lambda i: (i,)),
        ],
        out_shape=[
            jax.ShapeDtypeStruct((N, D), jnp.float32),
            jax.ShapeDtypeStruct((N,), jnp.float32),
        ],
    )(x, W_gcn, deg_parts)

    acc_parts = _make_spmm()(edge_index, g, _ZEROS_ND)

    eps = _eps_np()

    x_new, mu, std = pl.pallas_call(
        _mlp_body,
        grid=(_G,),
        in_specs=[
            pl.BlockSpec((NC, _R, D), lambda i: (0, i, 0)),
            pl.BlockSpec((_R, D), lambda i: (i, 0)),
            pl.BlockSpec((_R,), lambda i: (i,)),
            pl.BlockSpec((D,), lambda i: (0,)),
            pl.BlockSpec((DEPTH, D, D), lambda i: (0, 0, 0)),
            pl.BlockSpec((DEPTH, D), lambda i: (0, 0)),
            pl.BlockSpec((D, D), lambda i: (0, 0)),
            pl.BlockSpec((D,), lambda i: (0,)),
            pl.BlockSpec((D, D), lambda i: (0, 0)),
            pl.BlockSpec((D,), lambda i: (0,)),
            pl.BlockSpec((_R, D), lambda i: (i, 0)),
        ],
        out_specs=[
            pl.BlockSpec((_R, D), lambda i: (i, 0)),
            pl.BlockSpec((_R, D), lambda i: (i, 0)),
            pl.BlockSpec((_R, D), lambda i: (i, 0)),
        ],
        out_shape=[
            jax.ShapeDtypeStruct((N, D), jnp.float32),
            jax.ShapeDtypeStruct((N, D), jnp.float32),
            jax.ShapeDtypeStruct((N, D), jnp.float32),
        ],
    )(acc_parts, g, dinv, b_gcn, W_enc, b_enc, W_mu, b_mu, W_std, b_std, eps)

    return (x_new, mu, std)


# speedup vs baseline: 1.2746x; 1.1244x over previous
import functools

import jax
import jax.numpy as jnp
import numpy as np
from jax import lax
from jax.experimental import pallas as pl
from jax.experimental.pallas import tpu as pltpu
from jax.experimental.pallas import tpu_sc as plsc

N = 10000
E = 320000
D = 128
DEPTH = 5

NC = 2
NS = 16
NW = NC * NS
E_PER = E // NW
DEG_CH = 2000
EDGE_CH = 80
N_CHUNKS = E_PER // EDGE_CH
SP_CH = 96
NSETS = 4
NIDX = 8
SP_ITERS = E_PER // (NSETS * SP_CH)
SP_FULL = SP_ITERS * NSETS
SP_REM = E_PER - SP_FULL * SP_CH
ROWS_PER_TILE = N // NS

@functools.cache
def _make_deg():
    mesh = plsc.VectorSubcoreMesh(core_axis_name="c", subcore_axis_name="s",
                                  num_cores=NC, num_subcores=NS)
    return pl.kernel(
        _deg_body,
        out_type=jax.ShapeDtypeStruct((NC, N), jnp.float32),
        mesh=mesh,
        scratch_types=[
            pltpu.VMEM((E_PER,), jnp.int32),
            pltpu.VMEM((EDGE_CH,), jnp.float32),
            pltpu.VMEM_SHARED((N,), jnp.float32),
            pltpu.SemaphoreType.DMA,
        ],
        compiler_params=pltpu.CompilerParams(use_tc_tiling_on_sc=False),
    )


def _deg_body(ei_hbm, zeros1_hbm, out_hbm, didx_v, ones_v, deg_s, sem):
    cid = lax.axis_index("c")
    sid = lax.axis_index("s")
    wid = sid * NC + cid
    base = wid * E_PER

    def fill(i, _):
        ones_v[pl.ds(i * 16, 16)] = jnp.full((16,), 1.0, jnp.float32)
        return 0

    lax.fori_loop(0, EDGE_CH // 16, fill, 0)

    @pl.when(sid < 10)
    def _():
        pltpu.sync_copy(zeros1_hbm.at[pl.ds(sid * 1000, 1000)],
                        deg_s.at[pl.ds(sid * 1000, 1000)])

    pltpu.sync_copy(ei_hbm.at[1, pl.ds(base, E_PER)], didx_v)
    plsc.subcore_barrier()

    def chunk(c, _):
        pltpu.async_copy(
            ones_v, deg_s.at[didx_v.at[pl.ds(c * EDGE_CH, EDGE_CH)]], sem,
            add=True)
        return 0

    lax.fori_loop(0, N_CHUNKS, chunk, 0)
    pltpu.make_async_copy(ei_hbm.at[1, pl.ds(base, E_PER)], didx_v, sem).wait()
    plsc.subcore_barrier()

    @pl.when(sid < 10)
    def _():
        pltpu.sync_copy(deg_s.at[pl.ds(sid * 1000, 1000)],
                        out_hbm.at[cid, pl.ds(sid * 1000, 1000)])


@functools.cache
def _make_spmm():
    mesh = plsc.VectorSubcoreMesh(core_axis_name="c", subcore_axis_name="s",
                                  num_cores=NC, num_subcores=NS)
    return pl.kernel(
        _spmm_body,
        out_type=jax.ShapeDtypeStruct((NC, N, D), jnp.float32),
        mesh=mesh,
        scratch_types=(
            [pltpu.VMEM((SP_CH,), jnp.int32) for _ in range(NIDX)]
            + [pltpu.VMEM((SP_CH,), jnp.int32) for _ in range(NIDX)]
            + [pltpu.VMEM((SP_CH, D), jnp.float32) for _ in range(NSETS)]
            + [pltpu.VMEM_SHARED((N, D), jnp.float32)]
            + [pltpu.SemaphoreType.DMA for _ in range(2 * NSETS + 1)]
        ),
        compiler_params=pltpu.CompilerParams(use_tc_tiling_on_sc=False),
    )


def _spmm_body(ei_hbm, g_hbm, zeros_hbm, out_hbm, *refs):
    sidx = refs[0:NIDX]
    didx = refs[NIDX:2 * NIDX]
    rows = refs[2 * NIDX:2 * NIDX + NSETS]
    acc_s = refs[2 * NIDX + NSETS]
    semg = refs[2 * NIDX + NSETS + 1:2 * NIDX + NSETS + 1 + NSETS]
    sems = refs[2 * NIDX + NSETS + 1 + NSETS:2 * NIDX + 1 + 3 * NSETS]
    semi = refs[2 * NIDX + 1 + 3 * NSETS]

    cid = lax.axis_index("c")
    sid = lax.axis_index("s")
    wid = sid * NC + cid
    base = wid * E_PER
    row0 = sid * ROWS_PER_TILE

    pltpu.sync_copy(zeros_hbm.at[pl.ds(row0, ROWS_PER_TILE)],
                    acc_s.at[pl.ds(row0, ROWS_PER_TILE)])
    plsc.subcore_barrier()

    def stage_idx(k, c):
        pltpu.async_copy(ei_hbm.at[0, pl.ds(base + c * SP_CH, SP_CH)],
                         sidx[k], semi)
        pltpu.async_copy(ei_hbm.at[1, pl.ds(base + c * SP_CH, SP_CH)],
                         didx[k], semi)

    def wait_idx(k):
        pltpu.make_async_copy(ei_hbm.at[0, pl.ds(0, SP_CH)], sidx[k],
                              semi).wait()
        pltpu.make_async_copy(ei_hbm.at[0, pl.ds(0, SP_CH)], didx[k],
                              semi).wait()

    def wait_gather(r):
        pltpu.make_async_copy(zeros_hbm.at[pl.ds(0, SP_CH)], rows[r],
                              semg[r]).wait()

    def fire_scatter(r, k):
        pltpu.async_copy(rows[r], acc_s.at[didx[k]], sems[r], add=True)

    def drain_scatter(r):
        pltpu.make_async_copy(zeros_hbm.at[pl.ds(0, SP_CH)], rows[r],
                              sems[r]).wait()

    stage_idx(0, 0)

    def ring(i, _):
        c0 = i * NIDX
        for k in range(NIDX):
            c = c0 + k
            r = k % NSETS

            @pl.when(c >= NSETS)
            def _():
                drain_scatter(r)

            @pl.when(c + 1 < SP_FULL)
            def _():
                stage_idx((k + 1) % NIDX, c + 1)

            wait_idx(k)
            pltpu.async_copy(g_hbm.at[sidx[k]], rows[r], semg[r])
            rw = (k - 2) % NSETS
            kw = (k - 2) % NIDX

            @pl.when(c >= 2)
            def _():
                wait_gather(rw)
                fire_scatter(rw, kw)

        return 0

    lax.fori_loop(0, SP_FULL // NIDX, ring, 0)

    for c in (SP_FULL - 2, SP_FULL - 1):
        wait_gather(c % NSETS)
        fire_scatter(c % NSETS, c % NIDX)
    if SP_REM:
        eL = base + SP_FULL * SP_CH
        sL = SP_FULL % NSETS
        kL = 0
        drain_scatter(sL)
        pltpu.sync_copy(ei_hbm.at[0, pl.ds(eL, SP_REM)],
                        sidx[kL].at[pl.ds(0, SP_REM)])
        pltpu.sync_copy(ei_hbm.at[1, pl.ds(eL, SP_REM)],
                        didx[kL].at[pl.ds(0, SP_REM)])
        pltpu.async_copy(g_hbm.at[sidx[kL].at[pl.ds(0, SP_REM)]],
                         rows[sL].at[pl.ds(0, SP_REM)], semg[sL]).wait()
        pltpu.sync_copy(rows[sL].at[pl.ds(0, SP_REM)],
                        acc_s.at[didx[kL].at[pl.ds(0, SP_REM)]], add=True)
        for r in range(NSETS):
            if r != sL:
                drain_scatter(r)
    else:
        for r in range(NSETS):
            drain_scatter(r)
    plsc.subcore_barrier()
    pltpu.sync_copy(acc_s.at[pl.ds(row0, ROWS_PER_TILE)],
                    out_hbm.at[cid, pl.ds(row0, ROWS_PER_TILE)])


def _scale_body(x_ref, w_ref, degp_ref, g_ref, dinv_ref):
    h0 = jnp.dot(x_ref[...], w_ref[...], preferred_element_type=jnp.float32)
    deg = jnp.sum(degp_ref[...], axis=0) + 1.0
    dinv = lax.rsqrt(deg)
    g_ref[...] = h0 * dinv[:, None]
    dinv_ref[...] = dinv


def _mlp_body(accp_ref, g_ref, dinv_ref, bgcn_ref, wenc_ref, benc_ref,
              wmu_ref, bmu_ref, wstd_ref, bstd_ref, eps_ref,
              xn_ref, mu_ref, std_ref):
    acc = accp_ref[0] + accp_ref[1]
    h = dinv_ref[...][:, None] * (acc + g_ref[...]) + bgcn_ref[...][None, :]
    for i in range(DEPTH):
        z = jnp.dot(h, wenc_ref[i], preferred_element_type=jnp.float32)
        h = jax.nn.sigmoid(z + benc_ref[i][None, :])
    mu = jnp.dot(h, wmu_ref[...], preferred_element_type=jnp.float32)
    mu = mu + bmu_ref[...][None, :]
    s = jnp.dot(h, wstd_ref[...], preferred_element_type=jnp.float32)
    s = s + bstd_ref[...][None, :] - 5.0
    std = jnp.maximum(s, 0.0) + jnp.log1p(jnp.exp(-jnp.abs(s)))
    mu_ref[...] = mu
    std_ref[...] = std
    xn_ref[...] = mu + std * eps_ref[...]


_R = 1024
_G = (N + _R - 1) // _R


@functools.cache
def _eps_np():
    with jax.ensure_compile_time_eval():
        with jax.default_device(jax.devices("cpu")[0]):
            return np.asarray(jax.random.uniform(
                jax.random.key(42), (N, D), dtype=jnp.float32))


_ZEROS_ND = np.zeros((N, D), np.float32)
_ZEROS_N = np.zeros((N,), np.float32)


def kernel(x, edge_index, W_gcn, b_gcn, W_enc, b_enc, W_mu, b_mu, W_std,
           b_std):
    deg_parts = _make_deg()(edge_index, _ZEROS_N)

    g, dinv = pl.pallas_call(
        _scale_body,
        grid=(_G,),
        in_specs=[
            pl.BlockSpec((_R, D), lambda i: (i, 0)),
            pl.BlockSpec((D, D), lambda i: (0, 0)),
            pl.BlockSpec((NC, _R), lambda i: (0, i)),
        ],
        out_specs=[
            pl.BlockSpec((_R, D), lambda i: (i, 0)),
            pl.BlockSpec((_R,), lambda i: (i,)),
        ],
        out_shape=[
            jax.ShapeDtypeStruct((N, D), jnp.float32),
            jax.ShapeDtypeStruct((N,), jnp.float32),
        ],
    )(x, W_gcn, deg_parts)

    acc_parts = _make_spmm()(edge_index, g, _ZEROS_ND)

    eps = _eps_np()

    x_new, mu, std = pl.pallas_call(
        _mlp_body,
        grid=(_G,),
        in_specs=[
            pl.BlockSpec((NC, _R, D), lambda i: (0, i, 0)),
            pl.BlockSpec((_R, D), lambda i: (i, 0)),
            pl.BlockSpec((_R,), lambda i: (i,)),
            pl.BlockSpec((D,), lambda i: (0,)),
            pl.BlockSpec((DEPTH, D, D), lambda i: (0, 0, 0)),
            pl.BlockSpec((DEPTH, D), lambda i: (0, 0)),
            pl.BlockSpec((D, D), lambda i: (0, 0)),
            pl.BlockSpec((D,), lambda i: (0,)),
            pl.BlockSpec((D, D), lambda i: (0, 0)),
            pl.BlockSpec((D,), lambda i: (0,)),
            pl.BlockSpec((_R, D), lambda i: (i, 0)),
        ],
        out_specs=[
            pl.BlockSpec((_R, D), lambda i: (i, 0)),
            pl.BlockSpec((_R, D), lambda i: (i, 0)),
            pl.BlockSpec((_R, D), lambda i: (i, 0)),
        ],
        out_shape=[
            jax.ShapeDtypeStruct((N, D), jnp.float32),
            jax.ShapeDtypeStruct((N, D), jnp.float32),
            jax.ShapeDtypeStruct((N, D), jnp.float32),
        ],
    )(acc_parts, g, dinv, b_gcn, W_enc, b_enc, W_mu, b_mu, W_std, b_std, eps)

    return (x_new, mu, std)


# speedup vs baseline: 1.2751x; 1.0003x over previous
import functools

import jax
import jax.numpy as jnp
import numpy as np
from jax import lax
from jax.experimental import pallas as pl
from jax.experimental.pallas import tpu as pltpu
from jax.experimental.pallas import tpu_sc as plsc

N = 10000
E = 320000
D = 128
DEPTH = 5

NC = 2
NS = 16
NW = NC * NS
E_PER = E // NW
EDGE_CH = 80
N_CHUNKS = E_PER // EDGE_CH
SP_CH = 96
NSETS = 4
NIDX = 8
SP_ITERS = E_PER // (NSETS * SP_CH)
SP_FULL = SP_ITERS * NSETS
SP_REM = E_PER - SP_FULL * SP_CH
ROWS_PER_TILE = N // NS

@functools.cache
def _make_deg():
    mesh = plsc.VectorSubcoreMesh(core_axis_name="c", subcore_axis_name="s",
                                  num_cores=NC, num_subcores=NS)
    return pl.kernel(
        _deg_body,
        out_type=jax.ShapeDtypeStruct((NC, N), jnp.float32),
        mesh=mesh,
        scratch_types=[
            pltpu.VMEM((E_PER,), jnp.int32),
            pltpu.VMEM((EDGE_CH,), jnp.float32),
            pltpu.VMEM_SHARED((N,), jnp.float32),
            pltpu.SemaphoreType.DMA,
        ],
        compiler_params=pltpu.CompilerParams(use_tc_tiling_on_sc=False),
    )


def _deg_body(ei_hbm, zeros1_hbm, out_hbm, didx_v, ones_v, deg_s, sem):
    cid = lax.axis_index("c")
    sid = lax.axis_index("s")
    wid = sid * NC + cid
    base = wid * E_PER

    def fill(i, _):
        ones_v[pl.ds(i * 16, 16)] = jnp.full((16,), 1.0, jnp.float32)
        return 0

    lax.fori_loop(0, EDGE_CH // 16, fill, 0)

    @pl.when(sid < 10)
    def _():
        pltpu.sync_copy(zeros1_hbm.at[pl.ds(sid * 1000, 1000)],
                        deg_s.at[pl.ds(sid * 1000, 1000)])

    pltpu.sync_copy(ei_hbm.at[1, pl.ds(base, E_PER)], didx_v)
    plsc.subcore_barrier()

    def chunk(c, _):
        pltpu.async_copy(
            ones_v, deg_s.at[didx_v.at[pl.ds(c * EDGE_CH, EDGE_CH)]], sem,
            add=True)
        return 0

    lax.fori_loop(0, N_CHUNKS, chunk, 0)
    pltpu.make_async_copy(ei_hbm.at[1, pl.ds(base, E_PER)], didx_v, sem).wait()
    plsc.subcore_barrier()

    @pl.when(sid < 10)
    def _():
        pltpu.sync_copy(deg_s.at[pl.ds(sid * 1000, 1000)],
                        out_hbm.at[cid, pl.ds(sid * 1000, 1000)])


@functools.cache
def _make_spmm():
    mesh = plsc.VectorSubcoreMesh(core_axis_name="c", subcore_axis_name="s",
                                  num_cores=NC, num_subcores=NS)
    return pl.kernel(
        _spmm_body,
        out_type=jax.ShapeDtypeStruct((NC, N, D), jnp.float32),
        mesh=mesh,
        scratch_types=(
            [pltpu.VMEM((SP_CH,), jnp.int32) for _ in range(NIDX)]
            + [pltpu.VMEM((SP_CH,), jnp.int32) for _ in range(NIDX)]
            + [pltpu.VMEM((SP_CH, D), jnp.float32) for _ in range(NSETS)]
            + [pltpu.VMEM_SHARED((N, D), jnp.float32)]
            + [pltpu.SemaphoreType.DMA for _ in range(2 * NSETS + 1)]
        ),
        compiler_params=pltpu.CompilerParams(use_tc_tiling_on_sc=False),
    )


def _spmm_body(ei_hbm, g_hbm, zeros_hbm, out_hbm, *refs):
    sidx = refs[0:NIDX]
    didx = refs[NIDX:2 * NIDX]
    rows = refs[2 * NIDX:2 * NIDX + NSETS]
    acc_s = refs[2 * NIDX + NSETS]
    semg = refs[2 * NIDX + NSETS + 1:2 * NIDX + NSETS + 1 + NSETS]
    sems = refs[2 * NIDX + NSETS + 1 + NSETS:2 * NIDX + 1 + 3 * NSETS]
    semi = refs[2 * NIDX + 1 + 3 * NSETS]

    cid = lax.axis_index("c")
    sid = lax.axis_index("s")
    wid = sid * NC + cid
    base = wid * E_PER
    row0 = sid * ROWS_PER_TILE

    pltpu.sync_copy(zeros_hbm.at[pl.ds(row0, ROWS_PER_TILE)],
                    acc_s.at[pl.ds(row0, ROWS_PER_TILE)])
    plsc.subcore_barrier()

    def stage_idx(k, c):
        pltpu.async_copy(ei_hbm.at[0, pl.ds(base + c * SP_CH, SP_CH)],
                         sidx[k], semi)
        pltpu.async_copy(ei_hbm.at[1, pl.ds(base + c * SP_CH, SP_CH)],
                         didx[k], semi)

    def wait_idx(k):
        pltpu.make_async_copy(ei_hbm.at[0, pl.ds(0, SP_CH)], sidx[k],
                              semi).wait()
        pltpu.make_async_copy(ei_hbm.at[0, pl.ds(0, SP_CH)], didx[k],
                              semi).wait()

    def wait_gather(r):
        pltpu.make_async_copy(zeros_hbm.at[pl.ds(0, SP_CH)], rows[r],
                              semg[r]).wait()

    def fire_scatter(r, k):
        pltpu.async_copy(rows[r], acc_s.at[didx[k]], sems[r], add=True)

    def drain_scatter(r):
        pltpu.make_async_copy(zeros_hbm.at[pl.ds(0, SP_CH)], rows[r],
                              sems[r]).wait()

    stage_idx(0, 0)

    def ring(i, _):
        c0 = i * NIDX
        for k in range(NIDX):
            c = c0 + k
            r = k % NSETS

            @pl.when(c >= NSETS)
            def _():
                drain_scatter(r)

            @pl.when(c + 1 < SP_FULL)
            def _():
                stage_idx((k + 1) % NIDX, c + 1)

            wait_idx(k)
            pltpu.async_copy(g_hbm.at[sidx[k]], rows[r], semg[r])
            rw = (k - 2) % NSETS
            kw = (k - 2) % NIDX

            @pl.when(c >= 2)
            def _():
                wait_gather(rw)
                fire_scatter(rw, kw)

        return 0

    lax.fori_loop(0, SP_FULL // NIDX, ring, 0)

    for c in (SP_FULL - 2, SP_FULL - 1):
        wait_gather(c % NSETS)
        fire_scatter(c % NSETS, c % NIDX)
    if SP_REM:
        eL = base + SP_FULL * SP_CH
        sL = SP_FULL % NSETS
        kL = 0
        drain_scatter(sL)
        pltpu.sync_copy(ei_hbm.at[0, pl.ds(eL, SP_REM)],
                        sidx[kL].at[pl.ds(0, SP_REM)])
        pltpu.sync_copy(ei_hbm.at[1, pl.ds(eL, SP_REM)],
                        didx[kL].at[pl.ds(0, SP_REM)])
        pltpu.async_copy(g_hbm.at[sidx[kL].at[pl.ds(0, SP_REM)]],
                         rows[sL].at[pl.ds(0, SP_REM)], semg[sL]).wait()
        pltpu.sync_copy(rows[sL].at[pl.ds(0, SP_REM)],
                        acc_s.at[didx[kL].at[pl.ds(0, SP_REM)]], add=True)
        for r in range(NSETS):
            if r != sL:
                drain_scatter(r)
    else:
        for r in range(NSETS):
            drain_scatter(r)
    plsc.subcore_barrier()
    pltpu.sync_copy(acc_s.at[pl.ds(row0, ROWS_PER_TILE)],
                    out_hbm.at[cid, pl.ds(row0, ROWS_PER_TILE)])


def _scale_body(x_ref, w_ref, degp_ref, g_ref, dinv_ref):
    h0 = jnp.dot(x_ref[...], w_ref[...], preferred_element_type=jnp.float32)
    deg = jnp.sum(degp_ref[...], axis=0) + 1.0
    dinv = lax.rsqrt(deg)
    g_ref[...] = h0 * dinv[:, None]
    dinv_ref[...] = dinv


def _mlp_body(accp_ref, g_ref, dinv_ref, bgcn_ref, wenc_ref, benc_ref,
              wmu_ref, bmu_ref, wstd_ref, bstd_ref, eps_ref,
              xn_ref, mu_ref, std_ref):
    acc = accp_ref[0] + accp_ref[1]
    h = dinv_ref[...][:, None] * (acc + g_ref[...]) + bgcn_ref[...][None, :]
    for i in range(DEPTH):
        z = jnp.dot(h, wenc_ref[i], preferred_element_type=jnp.float32)
        h = jax.nn.sigmoid(z + benc_ref[i][None, :])
    mu = jnp.dot(h, wmu_ref[...], preferred_element_type=jnp.float32)
    mu = mu + bmu_ref[...][None, :]
    s = jnp.dot(h, wstd_ref[...], preferred_element_type=jnp.float32)
    s = s + bstd_ref[...][None, :] - 5.0
    std = jnp.maximum(s, 0.0) + jnp.log1p(jnp.exp(-jnp.abs(s)))
    mu_ref[...] = mu
    std_ref[...] = std
    xn_ref[...] = mu + std * eps_ref[...]


_R = 1024
_G = (N + _R - 1) // _R


@functools.cache
def _eps_np():
    with jax.ensure_compile_time_eval():
        with jax.default_device(jax.devices("cpu")[0]):
            return np.asarray(jax.random.uniform(
                jax.random.key(42), (N, D), dtype=jnp.float32))


_ZEROS_ND = np.zeros((N, D), np.float32)
_ZEROS_N = np.zeros((N,), np.float32)


def kernel(x, edge_index, W_gcn, b_gcn, W_enc, b_enc, W_mu, b_mu, W_std,
           b_std):
    deg_parts = _make_deg()(edge_index, _ZEROS_N)

    g, dinv = pl.pallas_call(
        _scale_body,
        grid=(_G,),
        in_specs=[
            pl.BlockSpec((_R, D), lambda i: (i, 0)),
            pl.BlockSpec((D, D), lambda i: (0, 0)),
            pl.BlockSpec((NC, _R), lambda i: (0, i)),
        ],
        out_specs=[
            pl.BlockSpec((_R, D), lambda i: (i, 0)),
            pl.BlockSpec((_R,), lambda i: (i,)),
        ],
        out_shape=[
            jax.ShapeDtypeStruct((N, D), jnp.float32),
            jax.ShapeDtypeStruct((N,), jnp.float32),
        ],
    )(x, W_gcn, deg_parts)

    acc_parts = _make_spmm()(edge_index, g, _ZEROS_ND)

    eps = _eps_np()

    x_new, mu, std = pl.pallas_call(
        _mlp_body,
        grid=(_G,),
        in_specs=[
            pl.BlockSpec((NC, _R, D), lambda i: (0, i, 0)),
            pl.BlockSpec((_R, D), lambda i: (i, 0)),
            pl.BlockSpec((_R,), lambda i: (i,)),
            pl.BlockSpec((D,), lambda i: (0,)),
            pl.BlockSpec((DEPTH, D, D), lambda i: (0, 0, 0)),
            pl.BlockSpec((DEPTH, D), lambda i: (0, 0)),
            pl.BlockSpec((D, D), lambda i: (0, 0)),
            pl.BlockSpec((D,), lambda i: (0,)),
            pl.BlockSpec((D, D), lambda i: (0, 0)),
            pl.BlockSpec((D,), lambda i: (0,)),
            pl.BlockSpec((_R, D), lambda i: (i, 0)),
        ],
        out_specs=[
            pl.BlockSpec((_R, D), lambda i: (i, 0)),
            pl.BlockSpec((_R, D), lambda i: (i, 0)),
            pl.BlockSpec((_R, D), lambda i: (i, 0)),
        ],
        out_shape=[
            jax.ShapeDtypeStruct((N, D), jnp.float32),
            jax.ShapeDtypeStruct((N, D), jnp.float32),
            jax.ShapeDtypeStruct((N, D), jnp.float32),
        ],
    )(acc_parts, g, dinv, b_gcn, W_enc, b_enc, W_mu, b_mu, W_std, b_std, eps)

    return (x_new, mu, std)
